# baseline (speedup 1.0000x reference)
import functools

import jax
import jax.numpy as jnp
from jax import lax
from jax.experimental import pallas as pl
from jax.experimental.pallas import tpu as pltpu
from jax.experimental.pallas import tpu_sc as plsc

N = 10000
E = 160000
F = 128
C = 128
N_PAD = 10240
N_FPAD = 10016
NC, NS = 2, 16
NW = NC * NS
L = 16
CHUNK = 128
CPT = 40
E_PAD = NW * CPT * CHUNK
NROWS = E_PAD // CHUNK
T_SIZE = N * N + 64
ROWS_PER_TILE = N_PAD // NS
NEG_E = 0.2
NEG_OUT = 0.3

_sc_mesh = plsc.VectorSubcoreMesh(
    core_axis_name="c", subcore_axis_name="s", num_cores=NC, num_subcores=NS
)
_sc_params = pltpu.CompilerParams(needs_layout_passes=False)


BN_A = 400

def _tc_project_body(x_ref, w_ref, asrc_ref, adst_ref,
                     h_ref, fs_ref, fd_ref, hsum_ref):
    xb = x_ref[...]
    hb = jnp.dot(xb, w_ref[...], preferred_element_type=jnp.float32)
    h_ref[...] = hb
    fs_ref[...] = jnp.sum(hb * asrc_ref[...], axis=1, keepdims=True)
    fd_ref[...] = jnp.sum(hb * adst_ref[...], axis=1, keepdims=True)

    @pl.when(pl.program_id(0) == 0)
    def _():
        hsum_ref[...] = jnp.zeros_like(hsum_ref)

    hsum_ref[...] += jnp.sum(hb, axis=0, keepdims=True)


def _tc_project(x, w, asrc2, adst2):
    return pl.pallas_call(
        _tc_project_body,
        grid=(N // BN_A,),
        in_specs=[
            pl.BlockSpec((BN_A, F), lambda i: (i, 0)),
            pl.BlockSpec((F, C), lambda i: (0, 0)),
            pl.BlockSpec((1, C), lambda i: (0, 0)),
            pl.BlockSpec((1, C), lambda i: (0, 0)),
        ],
        out_specs=[
            pl.BlockSpec((BN_A, C), lambda i: (i, 0)),
            pl.BlockSpec((BN_A, 1), lambda i: (i, 0)),
            pl.BlockSpec((BN_A, 1), lambda i: (i, 0)),
            pl.BlockSpec((1, C), lambda i: (0, 0)),
        ],
        out_shape=[
            jax.ShapeDtypeStruct((N, C), jnp.float32),
            jax.ShapeDtypeStruct((N, 1), jnp.float32),
            jax.ShapeDtypeStruct((N, 1), jnp.float32),
            jax.ShapeDtypeStruct((1, C), jnp.float32),
        ],
    )(x, w, asrc2, adst2)


@functools.partial(
    pl.kernel,
    out_type=jax.ShapeDtypeStruct((T_SIZE,), jnp.int32),
    mesh=_sc_mesh,
    scratch_types=[pltpu.VMEM((CPT, CHUNK), jnp.int32) for _ in range(4)]
    + [pltpu.SemaphoreType.DMA, pltpu.SemaphoreType.DMA],
    compiler_params=_sc_params,
)
def _sc_dedup(s2_hbm, d2_hbm, t_hbm, s2, d2, kslab, idslab, sem_in, sem_s):
    wid = lax.axis_index("c") * NS + lax.axis_index("s")
    c0 = wid * CPT
    pltpu.async_copy(s2_hbm.at[pl.ds(c0, CPT)], s2, sem_in)
    pltpu.async_copy(d2_hbm.at[pl.ds(c0, CPT)], d2, sem_in)
    pltpu.make_async_copy(s2_hbm.at[pl.ds(c0, CPT)], s2, sem_in).wait()
    pltpu.make_async_copy(d2_hbm.at[pl.ds(c0, CPT)], d2, sem_in).wait()

    @pl.loop(0, CPT)
    def _(g):
        for r in range(CHUNK // L):
            sl = pl.ds(r * L, L)
            kslab[g, sl] = s2[g, sl] * N + d2[g, sl]
            idslab[g, sl] = (c0 + g) * CHUNK + r * L + lax.iota(jnp.int32, L)

    @pl.loop(0, CPT // 8)
    def _(wv):
        for u in range(8):
            g = wv * 8 + u
            pltpu.async_copy(idslab.at[g], t_hbm.at[kslab.at[g]], sem_s)
        for u in range(8):
            pltpu.make_async_copy(idslab.at[0], t_hbm.at[kslab.at[0]],
                                  sem_s).wait()


WPW = 8
NWAVES = CPT // WPW

@functools.partial(
    pl.kernel,
    out_type=[
        jax.ShapeDtypeStruct((NC, N_PAD, C), jnp.float32),
        jax.ShapeDtypeStruct((NC, N_PAD), jnp.float32),
    ],
    mesh=_sc_mesh,
    scratch_types=[
        pltpu.VMEM((WPW, CHUNK), jnp.int32),
        pltpu.VMEM((WPW, CHUNK), jnp.int32),
        pltpu.VMEM((WPW, CHUNK), jnp.int32),
        pltpu.VMEM((WPW, CHUNK), jnp.int32),
        pltpu.VMEM((WPW, CHUNK), jnp.float32),
        pltpu.VMEM((WPW, CHUNK), jnp.float32),
        pltpu.VMEM((WPW, CHUNK), jnp.float32),
        pltpu.VMEM((CHUNK, C), jnp.float32),
        pltpu.VMEM((CHUNK, C), jnp.float32),
        pltpu.VMEM_SHARED((N_PAD, C), jnp.float32),
        pltpu.VMEM_SHARED((N_PAD,), jnp.float32),
        pltpu.SemaphoreType.DMA,
        pltpu.SemaphoreType.DMA,
        pltpu.SemaphoreType.DMA,
        pltpu.SemaphoreType.DMA,
        pltpu.SemaphoreType.DMA,
        pltpu.SemaphoreType.DMA,
        pltpu.SemaphoreType.DMA,
        pltpu.SemaphoreType.DMA,
    ],
    compiler_params=_sc_params,
)
def _sc_agg(h_hbm, fs_hbm, fd_hbm, s2_hbm, d2_hbm, t_hbm, acc_hbm, z_hbm,
            s2, d2, kslab, tslab, fsl, fdl, wslab, bufA, bufB,
            acc_sh, z_sh, sem_in, sem_t, sem_f, sem_z, sgA, sgB, ssA, ssB):
    cid = lax.axis_index("c")
    sid = lax.axis_index("s")
    wid = cid * NS + sid
    c0 = wid * CPT
    row0 = sid * ROWS_PER_TILE

    @pl.loop(0, CHUNK)
    def _(row):
        for q in range(C // L):
            bufA[row, pl.ds(q * L, L)] = jnp.zeros((L,), jnp.float32)

    @pl.loop(0, ROWS_PER_TILE // CHUNK)
    def _(jj):
        pltpu.sync_copy(bufA, acc_sh.at[pl.ds(row0 + jj * CHUNK, CHUNK)])
        pltpu.sync_copy(bufA.at[0], z_sh.at[pl.ds(row0 + jj * CHUNK, CHUNK)])

    plsc.subcore_barrier()

    def scale(buf, g):
        @pl.loop(0, CHUNK // L)
        def _(g2):
            w16 = wslab[g, pl.ds(g2 * L, L)]
            for r in range(L):
                ws = w16[r]
                row = g2 * L + r
                for q in range(C // L):
                    slq = pl.ds(q * L, L)
                    buf[row, slq] = buf[row, slq] * ws

    @pl.loop(0, NWAVES)
    def _(v):
        cb = c0 + v * WPW

        pltpu.async_copy(s2_hbm.at[pl.ds(cb, WPW)], s2, sem_in)
        pltpu.async_copy(d2_hbm.at[pl.ds(cb, WPW)], d2, sem_in)
        pltpu.make_async_copy(s2_hbm.at[pl.ds(cb, WPW)], s2, sem_in).wait()
        pltpu.make_async_copy(d2_hbm.at[pl.ds(cb, WPW)], d2, sem_in).wait()

        @pl.loop(0, WPW)
        def _(g):
            for r in range(CHUNK // L):
                sl = pl.ds(r * L, L)
                kslab[g, sl] = s2[g, sl] * N + d2[g, sl]

        @pl.loop(0, WPW)
        def _(g):
            pltpu.async_copy(t_hbm.at[kslab.at[g]], tslab.at[g], sem_t)
            pltpu.async_copy(fs_hbm.at[s2.at[g]], fsl.at[g], sem_f)
            pltpu.async_copy(fd_hbm.at[d2.at[g]], fdl.at[g], sem_f)

        @pl.loop(0, WPW)
        def _(g):
            pltpu.make_async_copy(t_hbm.at[kslab.at[0]], tslab.at[0],
                                  sem_t).wait()
            pltpu.make_async_copy(fs_hbm.at[s2.at[0]], fsl.at[0], sem_f).wait()
            pltpu.make_async_copy(fd_hbm.at[d2.at[0]], fdl.at[0], sem_f).wait()

        @pl.loop(0, WPW)
        def _(g):
            for r in range(CHUNK // L):
                sl = pl.ds(r * L, L)
                ids = (cb + g) * CHUNK + r * L + lax.iota(jnp.int32, L)
                logit = fsl[g, sl] + fdl[g, sl]
                logit = jnp.where(logit > 0, logit, logit * NEG_E)
                p = jnp.exp(logit)
                wslab[g, sl] = jnp.where(tslab[g, sl] == ids, p,
                                         jnp.float32(0.0))
            pltpu.async_copy(wslab.at[g], z_sh.at[s2.at[g]], sem_z, add=True)

        pltpu.async_copy(h_hbm.at[d2.at[0]], bufA, sgA)

        @pl.loop(0, WPW // 2)
        def _(i):
            a = 2 * i
            bb = 2 * i + 1
            pltpu.make_async_copy(h_hbm.at[d2.at[0]], bufA, sgA).wait()

            @pl.when(i > 0)
            def _():
                pltpu.make_async_copy(bufB, acc_sh.at[s2.at[0]], ssB).wait()

            pltpu.async_copy(h_hbm.at[d2.at[bb]], bufB, sgB)
            scale(bufA, a)
            pltpu.async_copy(bufA, acc_sh.at[s2.at[a]], ssA, add=True)
            pltpu.make_async_copy(h_hbm.at[d2.at[0]], bufB, sgB).wait()
            pltpu.make_async_copy(bufA, acc_sh.at[s2.at[0]], ssA).wait()

            @pl.when(i < WPW // 2 - 1)
            def _():
                pltpu.async_copy(h_hbm.at[d2.at[a + 2]], bufA, sgA)

            scale(bufB, bb)
            pltpu.async_copy(bufB, acc_sh.at[s2.at[bb]], ssB, add=True)

        pltpu.make_async_copy(bufB, acc_sh.at[s2.at[0]], ssB).wait()

        @pl.loop(0, WPW)
        def _(g):
            pltpu.make_async_copy(wslab.at[0], z_sh.at[s2.at[0]],
                                  sem_z).wait()

    plsc.subcore_barrier()
    pltpu.sync_copy(acc_sh.at[pl.ds(row0, ROWS_PER_TILE)],
                    acc_hbm.at[cid, pl.ds(row0, ROWS_PER_TILE)])
    pltpu.sync_copy(z_sh.at[pl.ds(row0, ROWS_PER_TILE)],
                    z_hbm.at[cid, pl.ds(row0, ROWS_PER_TILE)])


BN_D = 512

def _tc_finalize_body(acc_ref, z_ref, hsum_ref, b_ref, o_ref):
    a = acc_ref[0] + acc_ref[1]
    z = z_ref[0] + z_ref[1]
    nonempty = z > 0
    mean = hsum_ref[...] * jnp.float32(1.0 / N)
    val = jnp.where(nonempty, a / jnp.where(nonempty, z, jnp.float32(1.0)), mean)
    val = val + b_ref[...]
    o_ref[...] = jnp.where(val > 0, val, val * NEG_OUT)


def _tc_finalize(acc, z3, hsum, b2):
    return pl.pallas_call(
        _tc_finalize_body,
        grid=(N_PAD // BN_D,),
        in_specs=[
            pl.BlockSpec((NC, BN_D, C), lambda i: (0, i, 0)),
            pl.BlockSpec((NC, BN_D, 1), lambda i: (0, i, 0)),
            pl.BlockSpec((1, C), lambda i: (0, 0)),
            pl.BlockSpec((1, C), lambda i: (0, 0)),
        ],
        out_specs=pl.BlockSpec((BN_D, C), lambda i: (i, 0)),
        out_shape=jax.ShapeDtypeStruct((N_PAD, C), jnp.float32),
    )(acc, z3, hsum, b2)


def kernel(x, edge_index, W, a_src, a_dst, b):
    srcs2d = jnp.concatenate(
        [edge_index[0], jnp.full((E_PAD - E,), N, jnp.int32)]).reshape(NROWS, CHUNK)
    dsts2d = jnp.concatenate(
        [edge_index[1], jnp.zeros((E_PAD - E,), jnp.int32)]).reshape(NROWS, CHUNK)
    h, fs, fd, hsum = _tc_project(x, W, a_src.reshape(1, C), a_dst.reshape(1, C))
    fs_p = jnp.pad(fs.reshape(N), (0, N_FPAD - N))
    fd_p = jnp.pad(fd.reshape(N), (0, N_FPAD - N))
    t_tab = _sc_dedup(srcs2d, dsts2d)
    acc, z = _sc_agg(h, fs_p, fd_p, srcs2d, dsts2d, t_tab)
    out = _tc_finalize(acc, z.reshape(NC, N_PAD, 1), hsum, b.reshape(1, C))
    return out[:N]
